# baseline (speedup 1.0000x reference)
import jax
import jax.numpy as jnp
from jax.experimental import pallas as pl
from jax.experimental.pallas import tpu as pltpu

SCALING = 32.0 / 64.0
SCALE_B = 32.0
SCALE_H = 16.0
DESCALE = 1.0 / (SCALE_B * SCALE_H)


def _fused_kernel(x_ref, wbig_ref, bflat_ref, s_ref, t_ref, bias_ref,
                  out_ref, y_ref, *, nblocks):
    d_out = out_ref.shape[1]
    r = t_ref.shape[0]
    e = s_ref.shape[0]
    r_off = d_out
    a_off = d_out + e
    i = pl.program_id(0)

    if True:
        xb = x_ref[...].astype(jnp.bfloat16)
        y_ref[i % 2] = jnp.dot(xb, wbig_ref[...],
                               preferred_element_type=jnp.float32)

    if True:
        y = y_ref[(i - 1) % 2]
        h = jnp.maximum(y[:, a_off:a_off + r], 0.0)
        logits = y[:, r_off:r_off + e]
        m = jnp.max(logits, axis=-1, keepdims=True)
        p = jnp.exp(logits - m)
        wts = p / jnp.sum(p, axis=-1, keepdims=True)
        w_rep = jnp.dot(wts.astype(jnp.bfloat16), s_ref[...],
                        preferred_element_type=jnp.float32)
        h_tile = jnp.dot(h.astype(jnp.bfloat16), t_ref[...],
                         preferred_element_type=jnp.float32)
        hw = (w_rep * h_tile).astype(jnp.float8_e4m3fn)
        delta = jnp.dot(hw, bflat_ref[...],
                        preferred_element_type=jnp.float32)
        out_ref[...] = y[:, :d_out] + DESCALE * delta + bias_ref[...]


def kernel(x, base_W, base_b, A, B, router_W):
    import functools
    n, d_in = x.shape
    d_out = base_W.shape[0]
    e, _, r = B.shape
    tn = 512 if n % 512 == 0 else n
    nblocks = n // tn
    extra = ((e + r + 127) // 128) * 128

    w_t = base_W.T.astype(jnp.bfloat16)
    a_t = A.T.astype(jnp.bfloat16)
    r_t = router_W.T.astype(jnp.bfloat16)
    tail = jnp.pad(jnp.concatenate([r_t, a_t], axis=1),
                   ((0, 0), (0, extra - e - r)))
    w_big = jnp.concatenate([w_t, tail], axis=1)
    b_flat = ((SCALING * SCALE_B) * B.transpose(0, 2, 1).reshape(e * r, d_out)
              ).astype(jnp.float8_e4m3fn)
    bias = base_b.reshape(1, d_out)
    j = jnp.arange(e * r)
    s_pat = (SCALE_H * (j // r == jnp.arange(e)[:, None])
             ).astype(jnp.bfloat16)
    t_pat = (j % r == jnp.arange(r)[:, None]).astype(jnp.bfloat16)

    n_big = d_out + extra
    last = nblocks - 1
    return pl.pallas_call(
        functools.partial(_fused_kernel, nblocks=nblocks),
        grid=(nblocks + 1,),
        in_specs=[
            pl.BlockSpec((tn, d_in), lambda i: (jnp.minimum(i, last), 0)),
            pl.BlockSpec((d_in, n_big), lambda i: (0, 0)),
            pl.BlockSpec((e * r, d_out), lambda i: (0, 0)),
            pl.BlockSpec((e, e * r), lambda i: (0, 0)),
            pl.BlockSpec((r, e * r), lambda i: (0, 0)),
            pl.BlockSpec((1, d_out), lambda i: (0, 0)),
        ],
        out_specs=pl.BlockSpec((tn, d_out),
                               lambda i: (jnp.maximum(i - 1, 0), 0)),
        out_shape=jax.ShapeDtypeStruct((n, d_out), jnp.float32),
        scratch_shapes=[pltpu.VMEM((2, tn, n_big), jnp.float32)],
    )(x, w_big, b_flat, s_pat, t_pat, bias)


# speedup vs baseline: 1.1371x; 1.1371x over previous
import jax
import jax.numpy as jnp
from jax.experimental import pallas as pl
from jax.experimental.pallas import tpu as pltpu

SCALING = 32.0 / 64.0
SCALE_B = 32.0
SCALE_H = 16.0
DESCALE = 1.0 / (SCALE_B * SCALE_H)


def _fused_kernel(x_ref, wbig_ref, bflat_ref, s_ref, t_ref, bias_ref,
                  out_ref, y_ref, *, nblocks):
    d_out = out_ref.shape[1]
    r = t_ref.shape[0]
    e = s_ref.shape[0]
    r_off = d_out
    a_off = d_out + e
    i = pl.program_id(0)

    @pl.when(i > 0)
    def _consume():
        y = y_ref[(i - 1) % 2]
        h = jnp.maximum(y[:, a_off:a_off + r], 0.0)
        logits = y[:, r_off:r_off + e]
        m = jnp.max(logits, axis=-1, keepdims=True)
        p = jnp.exp(logits - m)
        wts = p / jnp.sum(p, axis=-1, keepdims=True)
        w_rep = jnp.dot(wts.astype(jnp.bfloat16), s_ref[...],
                        preferred_element_type=jnp.float32)
        h_tile = jnp.dot(h.astype(jnp.bfloat16), t_ref[...],
                         preferred_element_type=jnp.float32)
        hw = (w_rep * h_tile).astype(jnp.float8_e4m3fn)
        delta = jnp.dot(hw, bflat_ref[...],
                        preferred_element_type=jnp.float32)
        out_ref[...] = y[:, :d_out] + DESCALE * delta + bias_ref[...]


    @pl.when(i < nblocks)
    def _produce():
        xb = x_ref[...].astype(jnp.bfloat16)
        y_ref[i % 2] = jnp.dot(xb, wbig_ref[...],
                               preferred_element_type=jnp.float32)


def kernel(x, base_W, base_b, A, B, router_W):
    import functools
    n, d_in = x.shape
    d_out = base_W.shape[0]
    e, _, r = B.shape
    tn = 512 if n % 512 == 0 else n
    nblocks = n // tn
    extra = ((e + r + 127) // 128) * 128

    w_t = base_W.T.astype(jnp.bfloat16)
    a_t = A.T.astype(jnp.bfloat16)
    r_t = router_W.T.astype(jnp.bfloat16)
    tail = jnp.pad(jnp.concatenate([r_t, a_t], axis=1),
                   ((0, 0), (0, extra - e - r)))
    w_big = jnp.concatenate([w_t, tail], axis=1)
    b_flat = ((SCALING * SCALE_B) * B.transpose(0, 2, 1).reshape(e * r, d_out)
              ).astype(jnp.float8_e4m3fn)
    bias = base_b.reshape(1, d_out)
    j = jnp.arange(e * r)
    s_pat = (SCALE_H * (j // r == jnp.arange(e)[:, None])
             ).astype(jnp.bfloat16)
    t_pat = (j % r == jnp.arange(r)[:, None]).astype(jnp.bfloat16)

    n_big = d_out + extra
    last = nblocks - 1
    return pl.pallas_call(
        functools.partial(_fused_kernel, nblocks=nblocks),
        grid=(nblocks + 1,),
        in_specs=[
            pl.BlockSpec((tn, d_in), lambda i: (jnp.minimum(i, last), 0)),
            pl.BlockSpec((d_in, n_big), lambda i: (0, 0)),
            pl.BlockSpec((e * r, d_out), lambda i: (0, 0)),
            pl.BlockSpec((e, e * r), lambda i: (0, 0)),
            pl.BlockSpec((r, e * r), lambda i: (0, 0)),
            pl.BlockSpec((1, d_out), lambda i: (0, 0)),
        ],
        out_specs=pl.BlockSpec((tn, d_out),
                               lambda i: (jnp.maximum(i - 1, 0), 0)),
        out_shape=jax.ShapeDtypeStruct((n, d_out), jnp.float32),
        scratch_shapes=[pltpu.VMEM((2, tn, n_big), jnp.float32)],
    )(x, w_big, b_flat, s_pat, t_pat, bias)
